# baseline (speedup 1.0000x reference)
import functools

import jax
import jax.numpy as jnp
from jax import lax
from jax.experimental import pallas as pl
from jax.experimental.pallas import tpu as pltpu
from jax.experimental.pallas import tpu_sc as plsc

B, V, E, F = 16, 10000, 320000, 128
NC, NS, LN = 2, 16, 16
CW = 128
DQ = 2
JA = DQ // 2
EBLK = 2048
EP = 327680
EPT = EP // NS
NBLK = EPT // EBLK
NCH = EBLK // CW
NDT = 10
VPT = V // NDT
BPC = B // NC


def _tc_matmul(x2d, weight):
    R = 2000

    def body(x_ref, w_ref, a_ref, b_ref, c_ref):
        xb = x_ref[...]
        wa = w_ref[0] - w_ref[2]
        wb = w_ref[1]
        wc = 2.0 * w_ref[2]
        a_ref[...] = jnp.dot(xb, wa, preferred_element_type=jnp.float32)
        b_ref[...] = jnp.dot(xb, wb, preferred_element_type=jnp.float32)
        c_ref[...] = jnp.dot(xb, wc, preferred_element_type=jnp.float32)

    outs = pl.pallas_call(
        body,
        grid=(B * V // R,),
        in_specs=[
            pl.BlockSpec((R, F), lambda i: (i, 0)),
            pl.BlockSpec((3, F, F), lambda i: (0, 0, 0)),
        ],
        out_specs=[pl.BlockSpec((R, F), lambda i: (i, 0))] * 3,
        out_shape=[jax.ShapeDtypeStruct((B * V, F), jnp.float32)] * 3,
    )(x2d, weight)
    return outs


def _sc_spmm(init_flat, src_flat, packed_rc, w_flat):
    mesh = plsc.VectorSubcoreMesh(core_axis_name="c", subcore_axis_name="s")

    @functools.partial(
        pl.kernel,
        out_type=jax.ShapeDtypeStruct((B * V, F), jnp.float32),
        mesh=mesh,
        scratch_types=[
            pltpu.VMEM_SHARED((V, F), jnp.float32),
            pltpu.VMEM((EBLK,), jnp.int32),
            pltpu.VMEM((EBLK,), jnp.int32),
            pltpu.VMEM((EBLK,), jnp.float32),
            pltpu.VMEM((EBLK,), jnp.float32),
            pltpu.VMEM((EBLK,), jnp.int32),
            pltpu.VMEM((NCH, CW), jnp.int32),
            pltpu.VMEM((DQ, CW, F), jnp.float32),
        ] + [pltpu.SemaphoreType.DMA] * (2 * DQ + 2),
    )
    def k(init_hbm, src_hbm, rc_hbm, w_hbm, out_hbm,
          acc, pbuf0, pbuf1, wbuf0, wbuf1, cbuf, rbuf, gbuf, *sems):
        gsem = sems[:DQ]
        ssem = sems[DQ:2 * DQ]
        msem = sems[2 * DQ:]
        pbufs = (pbuf0, pbuf1)
        wbufs = (wbuf0, wbuf1)
        c = lax.axis_index("c")
        s = lax.axis_index("s")

        def md_issue(q, par):
            eb = s * EPT + q * EBLK
            pltpu.async_copy(rc_hbm.at[pl.ds(eb, EBLK)], pbufs[par], msem[par])
            pltpu.async_copy(w_hbm.at[pl.ds(eb, EBLK)], wbufs[par], msem[par])

        def md_wait(par):
            pltpu.make_async_copy(rc_hbm.at[pl.ds(0, EBLK)], pbufs[par],
                                  msem[par]).wait()
            pltpu.make_async_copy(w_hbm.at[pl.ds(0, EBLK)], wbufs[par],
                                  msem[par]).wait()

        def g_issue(j, d):
            pltpu.async_copy(src_hbm.at[cbuf.at[pl.ds(j * CW, CW)]],
                             gbuf.at[d], gsem[d])

        def g_wait(d):
            pltpu.make_async_copy(src_hbm.at[pl.ds(0, CW)], gbuf.at[d],
                                  gsem[d]).wait()

        def s_issue(j, d):
            pltpu.async_copy(gbuf.at[d], acc.at[rbuf.at[j]], ssem[d], add=True)

        def s_wait(d):
            pltpu.make_async_copy(gbuf.at[d], acc.at[pl.ds(0, CW)],
                                  ssem[d]).wait()

        def multiply(j, d, par):
            gb = gbuf.at[d]

            @pl.loop(0, CW // LN)
            def _grp(g):
                w16 = wbufs[par][pl.ds(j * CW + g * LN, LN)]
                for el in range(LN):
                    w = w16[el]
                    e = g * LN + el
                    for u in range(F // LN):
                        sl = pl.ds(u * LN, LN)
                        gb[e, sl] = gb[e, sl] * w

        def block(q, par, bv):
            md_wait(par)
            md_issue(lax.rem(q + 1, NBLK), 1 - par)

            pb = pbufs[par]

            @pl.loop(0, NCH)
            def _unpack(j):
                for u in range(CW // LN):
                    sl = pl.ds(j * CW + u * LN, LN)
                    v = pb[sl]
                    cbuf[sl] = (v & 0xFFFF) + bv
                    rbuf[j, pl.ds(u * LN, LN)] = lax.shift_right_logical(v, 16)

            for d in range(JA):
                g_issue(d, d)

            @pl.loop(0, NCH // DQ)
            def _steady(h):
                for d in range(DQ):
                    j = h * DQ + d
                    d2 = (d + JA) % DQ
                    g_wait(d)
                    multiply(j, d, par)
                    s_issue(j, d)

                    @pl.when(j >= JA)
                    def _():
                        s_wait(d2)

                    @pl.when(j + JA < NCH)
                    def _():
                        g_issue(j + JA, d2)

            for d in range(JA):
                s_wait((NCH - JA + d) % DQ)

        md_issue(0, 0)

        @pl.loop(0, BPC)
        def _per_b(i):
            bv = (c * BPC + i) * V

            @pl.when(s < NDT)
            def _():
                pltpu.sync_copy(init_hbm.at[pl.ds(bv + s * VPT, VPT)],
                                acc.at[pl.ds(s * VPT, VPT)])

            plsc.subcore_barrier()

            @pl.loop(0, NBLK // 2)
            def _blocks(hb):
                block(2 * hb, 0, bv)
                block(2 * hb + 1, 1, bv)

            plsc.subcore_barrier()

            @pl.when(s < NDT)
            def _():
                pltpu.sync_copy(acc.at[pl.ds(s * VPT, VPT)],
                                out_hbm.at[pl.ds(bv + s * VPT, VPT)])

            plsc.subcore_barrier()

        md_wait(0)

    return k(init_flat, src_flat, packed_rc, w_flat)


def _tc_stats(z2d):
    R = 2000

    def body(z_ref, o_ref):
        @pl.when(pl.program_id(0) == 0)
        def _():
            o_ref[...] = jnp.zeros_like(o_ref)

        zb = z_ref[...]
        o_ref[0:1, :] += jnp.sum(zb, axis=0, keepdims=True)
        o_ref[1:2, :] += jnp.sum(zb * zb, axis=0, keepdims=True)

    return pl.pallas_call(
        body,
        grid=(B * V // R,),
        in_specs=[pl.BlockSpec((R, F), lambda i: (i, 0))],
        out_specs=pl.BlockSpec((8, F), lambda i: (0, 0)),
        out_shape=jax.ShapeDtypeStruct((8, F), jnp.float32),
    )(z2d)


def _tc_bn(z3d, stats, gamma2d, beta2d):
    def body(z_ref, st_ref, g_ref, bt_ref, o_ref):
        n = float(B * V)
        mean = st_ref[0, :] / n
        var = st_ref[1, :] / n - mean * mean
        inv = lax.rsqrt(var + 1e-5)
        scale = g_ref[0, :] * inv
        shift = bt_ref[0, :] - mean * scale
        y = z_ref[0] * scale[None, :] + shift[None, :]
        o_ref[0] = y.T

    return pl.pallas_call(
        body,
        grid=(B,),
        in_specs=[
            pl.BlockSpec((1, V, F), lambda i: (i, 0, 0)),
            pl.BlockSpec((8, F), lambda i: (0, 0)),
            pl.BlockSpec((1, F), lambda i: (0, 0)),
            pl.BlockSpec((1, F), lambda i: (0, 0)),
        ],
        out_specs=pl.BlockSpec((1, F, V), lambda i: (i, 0, 0)),
        out_shape=jax.ShapeDtypeStruct((B, F, V), jnp.float32),
    )(z3d, stats, gamma2d, beta2d)


def kernel(x, edge_row, edge_col, edge_weight, weight, bias, gamma, beta):
    del bias
    x2d = x.reshape(B * V, F)
    y_a, y_b, y_c = _tc_matmul(x2d, weight)

    packed_rc = (edge_row.astype(jnp.int32) << 16) | edge_col.astype(jnp.int32)
    packed_rc = jnp.pad(packed_rc, (0, EP - E))
    w_pad = jnp.pad(edge_weight, (0, EP - E))
    t = _sc_spmm(y_b, y_c, packed_rc, w_pad)
    z = _sc_spmm(y_a, t, packed_rc, w_pad)

    stats = _tc_stats(z)
    return _tc_bn(z.reshape(B, V, F), stats,
                  gamma.reshape(1, F), beta.reshape(1, F))


# speedup vs baseline: 1.1819x; 1.1819x over previous
import functools

import jax
import jax.numpy as jnp
from jax import lax
from jax.experimental import pallas as pl
from jax.experimental.pallas import tpu as pltpu
from jax.experimental.pallas import tpu_sc as plsc

B, V, E, F = 16, 10000, 320000, 128
NC, NS, LN = 2, 16, 16
CW = 32
DQ = 8
JA = DQ // 2
EBLK = 1024
EP = 327680
EPT = EP // NS
NBLK = EPT // EBLK
NCH = EBLK // CW
NDT = 10
VPT = V // NDT
BPC = B // NC


def _tc_matmul(x2d, weight):
    R = 2000

    def body(x_ref, w_ref, a_ref, b_ref, c_ref):
        xb = x_ref[...]
        wa = w_ref[0] - w_ref[2]
        wb = w_ref[1]
        wc = 2.0 * w_ref[2]
        a_ref[...] = jnp.dot(xb, wa, preferred_element_type=jnp.float32)
        b_ref[...] = jnp.dot(xb, wb, preferred_element_type=jnp.float32)
        c_ref[...] = jnp.dot(xb, wc, preferred_element_type=jnp.float32)

    outs = pl.pallas_call(
        body,
        grid=(B * V // R,),
        in_specs=[
            pl.BlockSpec((R, F), lambda i: (i, 0)),
            pl.BlockSpec((3, F, F), lambda i: (0, 0, 0)),
        ],
        out_specs=[pl.BlockSpec((R, F), lambda i: (i, 0))] * 3,
        out_shape=[jax.ShapeDtypeStruct((B * V, F), jnp.float32)] * 3,
    )(x2d, weight)
    return outs


def _sc_spmm(init_flat, src_flat, packed_rc, w_flat):
    mesh = plsc.VectorSubcoreMesh(core_axis_name="c", subcore_axis_name="s")

    @functools.partial(
        pl.kernel,
        out_type=jax.ShapeDtypeStruct((B * V, F), jnp.float32),
        mesh=mesh,
        scratch_types=[
            pltpu.VMEM_SHARED((V, F), jnp.float32),
            pltpu.VMEM((EBLK,), jnp.int32),
            pltpu.VMEM((EBLK,), jnp.int32),
            pltpu.VMEM((EBLK,), jnp.float32),
            pltpu.VMEM((EBLK,), jnp.float32),
            pltpu.VMEM((EBLK,), jnp.int32),
            pltpu.VMEM((NCH, CW), jnp.int32),
            pltpu.VMEM((DQ, CW, F), jnp.float32),
        ] + [pltpu.SemaphoreType.DMA] * (2 * DQ + 2),
    )
    def k(init_hbm, src_hbm, rc_hbm, w_hbm, out_hbm,
          acc, pbuf0, pbuf1, wbuf0, wbuf1, cbuf, rbuf, gbuf, *sems):
        gsem = sems[:DQ]
        ssem = sems[DQ:2 * DQ]
        msem = sems[2 * DQ:]
        pbufs = (pbuf0, pbuf1)
        wbufs = (wbuf0, wbuf1)
        c = lax.axis_index("c")
        s = lax.axis_index("s")

        def md_issue(q, par):
            eb = s * EPT + q * EBLK
            pltpu.async_copy(rc_hbm.at[pl.ds(eb, EBLK)], pbufs[par], msem[par])
            pltpu.async_copy(w_hbm.at[pl.ds(eb, EBLK)], wbufs[par], msem[par])

        def md_wait(par):
            pltpu.make_async_copy(rc_hbm.at[pl.ds(0, EBLK)], pbufs[par],
                                  msem[par]).wait()
            pltpu.make_async_copy(w_hbm.at[pl.ds(0, EBLK)], wbufs[par],
                                  msem[par]).wait()

        def g_issue(j, d):
            pltpu.async_copy(src_hbm.at[cbuf.at[pl.ds(j * CW, CW)]],
                             gbuf.at[d], gsem[d])

        def g_wait(d):
            pltpu.make_async_copy(src_hbm.at[pl.ds(0, CW)], gbuf.at[d],
                                  gsem[d]).wait()

        def s_issue(j, d):
            pltpu.async_copy(gbuf.at[d], acc.at[rbuf.at[j]], ssem[d], add=True)

        def s_wait(d):
            pltpu.make_async_copy(gbuf.at[d], acc.at[pl.ds(0, CW)],
                                  ssem[d]).wait()

        def multiply(j, d, par):
            gb = gbuf.at[d]

            @pl.loop(0, CW // LN)
            def _grp(g):
                w16 = wbufs[par][pl.ds(j * CW + g * LN, LN)]
                for el in range(LN):
                    w = w16[el]
                    e = g * LN + el
                    for u in range(F // LN):
                        sl = pl.ds(u * LN, LN)
                        gb[e, sl] = gb[e, sl] * w

        def block(q, par, bv):
            md_wait(par)
            md_issue(lax.rem(q + 1, NBLK), 1 - par)

            pb = pbufs[par]

            @pl.loop(0, NCH)
            def _unpack(j):
                for u in range(CW // LN):
                    sl = pl.ds(j * CW + u * LN, LN)
                    v = pb[sl]
                    cbuf[sl] = (v & 0xFFFF) + bv
                    rbuf[j, pl.ds(u * LN, LN)] = lax.shift_right_logical(v, 16)

            for d in range(JA):
                g_issue(d, d)

            @pl.loop(0, NCH // DQ)
            def _steady(h):
                for d in range(DQ):
                    j = h * DQ + d
                    d2 = (d + JA) % DQ
                    g_wait(d)
                    multiply(j, d, par)
                    s_issue(j, d)

                    @pl.when(j >= JA)
                    def _():
                        s_wait(d2)

                    @pl.when(j + JA < NCH)
                    def _():
                        g_issue(j + JA, d2)

            for d in range(JA):
                s_wait((NCH - JA + d) % DQ)

        md_issue(0, 0)

        @pl.loop(0, BPC)
        def _per_b(i):
            bv = (c * BPC + i) * V

            @pl.when(s < NDT)
            def _():
                pltpu.sync_copy(init_hbm.at[pl.ds(bv + s * VPT, VPT)],
                                acc.at[pl.ds(s * VPT, VPT)])

            plsc.subcore_barrier()

            @pl.loop(0, NBLK // 2)
            def _blocks(hb):
                block(2 * hb, 0, bv)
                block(2 * hb + 1, 1, bv)

            plsc.subcore_barrier()

            @pl.when(s < NDT)
            def _():
                pltpu.sync_copy(acc.at[pl.ds(s * VPT, VPT)],
                                out_hbm.at[pl.ds(bv + s * VPT, VPT)])

            plsc.subcore_barrier()

        md_wait(0)

    return k(init_flat, src_flat, packed_rc, w_flat)


def _tc_stats(z2d):
    R = 2000

    def body(z_ref, o_ref):
        @pl.when(pl.program_id(0) == 0)
        def _():
            o_ref[...] = jnp.zeros_like(o_ref)

        zb = z_ref[...]
        o_ref[0:1, :] += jnp.sum(zb, axis=0, keepdims=True)
        o_ref[1:2, :] += jnp.sum(zb * zb, axis=0, keepdims=True)

    return pl.pallas_call(
        body,
        grid=(B * V // R,),
        in_specs=[pl.BlockSpec((R, F), lambda i: (i, 0))],
        out_specs=pl.BlockSpec((8, F), lambda i: (0, 0)),
        out_shape=jax.ShapeDtypeStruct((8, F), jnp.float32),
    )(z2d)


def _tc_bn(z3d, stats, gamma2d, beta2d):
    def body(z_ref, st_ref, g_ref, bt_ref, o_ref):
        n = float(B * V)
        mean = st_ref[0, :] / n
        var = st_ref[1, :] / n - mean * mean
        inv = lax.rsqrt(var + 1e-5)
        scale = g_ref[0, :] * inv
        shift = bt_ref[0, :] - mean * scale
        y = z_ref[0] * scale[None, :] + shift[None, :]
        o_ref[0] = y.T

    return pl.pallas_call(
        body,
        grid=(B,),
        in_specs=[
            pl.BlockSpec((1, V, F), lambda i: (i, 0, 0)),
            pl.BlockSpec((8, F), lambda i: (0, 0)),
            pl.BlockSpec((1, F), lambda i: (0, 0)),
            pl.BlockSpec((1, F), lambda i: (0, 0)),
        ],
        out_specs=pl.BlockSpec((1, F, V), lambda i: (i, 0, 0)),
        out_shape=jax.ShapeDtypeStruct((B, F, V), jnp.float32),
    )(z3d, stats, gamma2d, beta2d)


def kernel(x, edge_row, edge_col, edge_weight, weight, bias, gamma, beta):
    del bias
    x2d = x.reshape(B * V, F)
    y_a, y_b, y_c = _tc_matmul(x2d, weight)

    packed_rc = (edge_row.astype(jnp.int32) << 16) | edge_col.astype(jnp.int32)
    packed_rc = jnp.pad(packed_rc, (0, EP - E))
    w_pad = jnp.pad(edge_weight, (0, EP - E))
    t = _sc_spmm(y_b, y_c, packed_rc, w_pad)
    z = _sc_spmm(y_a, t, packed_rc, w_pad)

    stats = _tc_stats(z)
    return _tc_bn(z.reshape(B, V, F), stats,
                  gamma.reshape(1, F), beta.reshape(1, F))
